# Initial kernel scaffold; baseline (speedup 1.0000x reference)
import jax
import jax.numpy as jnp
from jax.experimental import pallas as pl


def kernel(q, k, v):
    raise NotImplementedError("write your pallas kernel here")


# speedup vs baseline: 4.6840x; 4.6840x over previous
import functools

import jax
import jax.numpy as jnp
import numpy as np
from jax.experimental import pallas as pl

SEGMENT_LENGTH = 2048
DILATION_RATE = 2


def _seg_attn_kernel(q_ref, k_ref, v_ref, o_ref, *, off, scale):
    seg = q_ref.shape[0]
    n = seg // DILATION_RATE
    sl = pl.ds(off, n, DILATION_RATE)
    qs = q_ref[sl, :] * scale
    ks = k_ref[sl, :]
    vs = v_ref[sl, :]
    s = jax.lax.dot_general(
        qs, ks, (((1,), (1,)), ((), ())), preferred_element_type=jnp.float32
    )
    m = jnp.max(s, axis=-1, keepdims=True)
    p = jnp.exp(s - m)
    l = jnp.sum(p, axis=-1, keepdims=True)
    o = jax.lax.dot_general(
        p, vs, (((1,), (0,)), ((), ())), preferred_element_type=jnp.float32
    )
    o = o / l
    o_ref[...] = jnp.zeros_like(o_ref)
    o_ref[sl, :] = o


def _segment_call(q, k, v, seg_idx, interpret=False):
    BH, M, D = q.shape
    off = seg_idx % DILATION_RATE
    scale = 1.0 / np.sqrt(float(D))
    block = (None, SEGMENT_LENGTH, D)
    idx_map = lambda bh: (bh, seg_idx, 0)
    return pl.pallas_call(
        functools.partial(_seg_attn_kernel, off=off, scale=scale),
        grid=(BH,),
        in_specs=[pl.BlockSpec(block, idx_map) for _ in range(3)],
        out_specs=pl.BlockSpec(block, lambda bh: (bh, 0, 0)),
        out_shape=jax.ShapeDtypeStruct((BH, SEGMENT_LENGTH, D), q.dtype),
        interpret=interpret,
    )(q, k, v)


@jax.jit
def kernel(q, k, v):
    B, H, M, D = q.shape
    qf = q.reshape(B * H, M, D)
    kf = k.reshape(B * H, M, D)
    vf = v.reshape(B * H, M, D)
    num_segments = M // SEGMENT_LENGTH
    outs = [
        _segment_call(qf, kf, vf, s) for s in range(num_segments)
    ]
    return jnp.concatenate(outs, axis=1).reshape(B, H, M, D)
